# baseline (speedup 1.0000x reference)
import functools

import jax
import jax.numpy as jnp
from jax.experimental import pallas as pl
from jax.experimental.pallas import tpu as pltpu

_SEG_TILE = 128
_MAX_TILES = 4


def _body(t0_ref, nt_ref, flat_ref, lo_ref, hi_ref, inv_ref, we_ref, wc_ref,
          out_ref, acc_ref, *, block_tok, num_blocks, num_sents):
    b = pl.program_id(0)

    @pl.when(b == 0)
    def _():
        acc_ref[...] = jnp.zeros_like(acc_ref)

    fb = flat_ref[...].astype(jnp.bfloat16)
    t = (jax.lax.broadcasted_iota(jnp.int32, (_SEG_TILE, block_tok), 1)
         + b * block_tok)
    t0 = t0_ref[b]
    nt = nt_ref[b]
    for j in range(_MAX_TILES):
        @pl.when(j < nt)
        def _():
            w = pl.multiple_of((t0 + j) * _SEG_TILE, _SEG_TILE)
            lo = lo_ref[pl.ds(w, _SEG_TILE), :]
            hi = hi_ref[pl.ds(w, _SEG_TILE), :]
            onehot = jnp.logical_and(t >= lo, t < hi).astype(jnp.bfloat16)
            part = jax.lax.dot_general(
                onehot, fb, (((1,), (0,)), ((), ())),
                preferred_element_type=jnp.float32)
            acc_ref[pl.ds(w, _SEG_TILE), :] += part

    @pl.when(b == num_blocks - 1)
    def _():
        mean = acc_ref[0:num_sents, :] * inv_ref[...]
        combined = jax.lax.dot_general(
            wc_ref[...], we_ref[...], (((1,), (0,)), ((), ())),
            precision=jax.lax.Precision.HIGHEST,
            preferred_element_type=jnp.float32)
        logits = jax.lax.dot_general(
            mean, combined, (((1,), (1,)), ((), ())),
            precision=jax.lax.Precision.HIGHEST,
            preferred_element_type=jnp.float32)
        m = jnp.max(logits, axis=-1, keepdims=True)
        sh = logits - m
        lse = jnp.log(jnp.sum(jnp.exp(sh), axis=-1, keepdims=True))
        out_ref[...] = sh - lse


def kernel(flat, cu_seqlens, W_e, W_c):
    total_tok, emb = flat.shape
    num_sents = cu_seqlens.shape[0] - 1
    ntags = W_c.shape[0]
    seg_pad = num_sents + _SEG_TILE

    cu = cu_seqlens.astype(jnp.int32)
    big = jnp.int32(2**30)
    cu_lo = jnp.full((seg_pad, 1), big, jnp.int32).at[:num_sents, 0].set(cu[:-1])
    cu_hi = jnp.full((seg_pad, 1), big, jnp.int32).at[:num_sents, 0].set(cu[1:])
    inv = 1.0 / jnp.maximum(cu[1:] - cu[:-1], 1).astype(jnp.float32)
    inv = inv.reshape(num_sents, 1)

    block_tok = 2048
    num_blocks = total_tok // block_tok

    starts = jnp.arange(num_blocks, dtype=jnp.int32) * block_tok
    first_seg = jnp.clip(
        jnp.searchsorted(cu, starts, side="right").astype(jnp.int32) - 1,
        0, num_sents - 1)
    last_seg = jnp.clip(
        jnp.searchsorted(cu, starts + (block_tok - 1), side="right")
        .astype(jnp.int32) - 1, 0, num_sents - 1)
    tile0 = first_seg // _SEG_TILE
    ntiles = last_seg // _SEG_TILE - tile0 + 1

    body = functools.partial(_body, block_tok=block_tok,
                             num_blocks=num_blocks, num_sents=num_sents)

    out = pl.pallas_call(
        body,
        grid_spec=pltpu.PrefetchScalarGridSpec(
            num_scalar_prefetch=2,
            grid=(num_blocks,),
            in_specs=[
                pl.BlockSpec((block_tok, emb), lambda b, *_: (b, 0)),
                pl.BlockSpec((seg_pad, 1), lambda b, *_: (0, 0)),
                pl.BlockSpec((seg_pad, 1), lambda b, *_: (0, 0)),
                pl.BlockSpec((num_sents, 1), lambda b, *_: (0, 0)),
                pl.BlockSpec((emb, emb), lambda b, *_: (0, 0)),
                pl.BlockSpec((ntags, emb), lambda b, *_: (0, 0)),
            ],
            out_specs=pl.BlockSpec((num_sents, ntags), lambda b, *_: (0, 0)),
            scratch_shapes=[pltpu.VMEM((seg_pad, emb), jnp.float32)],
        ),
        out_shape=jax.ShapeDtypeStruct((num_sents, ntags), jnp.float32),
    )(tile0, ntiles, flat, cu_lo, cu_hi, inv, W_e, W_c)
    return out
